# baseline (speedup 1.0000x reference)
import jax
import jax.numpy as jnp
from jax import lax
from jax.experimental import pallas as pl
from jax.experimental.pallas import tpu as pltpu
from jax.experimental.pallas import tpu_sc as plsc

LANES = 16
W = 16
NB = 8
L = 7


def kernel(input_ids, tok_embed, pos_embed):
    B, S = input_ids.shape
    H = tok_embed.shape[1]
    n = B * S
    NW = 32
    P = S // NW
    NCH = P // W
    NIT = NCH * B
    ids = input_ids.reshape(n).astype(jnp.int32)

    mesh = plsc.VectorSubcoreMesh(core_axis_name="c", subcore_axis_name="s")

    @jax.jit
    def run(ids, tok, pos):
        @pl.kernel(
            out_type=jax.ShapeDtypeStruct((n, H), jnp.float32),
            mesh=mesh,
            scratch_types=[pltpu.VMEM((B * P,), jnp.int32)]
            + [pltpu.VMEM((W, H), jnp.float32)] * 2
            + [pltpu.VMEM((W, H), jnp.float32)] * NB
            + [pltpu.SemaphoreType.DMA] * (2 + 2 * NB),
        )
        def emb_kernel(ids_hbm, tok_hbm, pos_hbm, out_hbm, idx_v, *rest):
            pos_v = rest[0:2]
            rows = rest[2:2 + NB]
            psem = rest[2 + NB:4 + NB]
            gsem = rest[4 + NB:4 + 2 * NB]
            ssem = rest[4 + 2 * NB:]
            wid = lax.axis_index("s") * 2 + lax.axis_index("c")
            pbase = wid * P

            def gather_start(j, b, ch):
                pltpu.async_copy(
                    tok_hbm.at[idx_v.at[pl.ds(b * P + ch * W, W)]],
                    rows[j], gsem[j])

            def pos_start(p, ch):
                pltpu.async_copy(pos_hbm.at[pl.ds(pbase + ch * W, W)],
                                 pos_v[p], psem[p])

            def drain(dst, sem):
                pltpu.make_async_copy(pos_hbm.at[pl.ds(0, W)], dst, sem).wait()

            idx_cps = [
                pltpu.async_copy(ids_hbm.at[pl.ds(b * S + pbase, P)],
                                 idx_v.at[pl.ds(b * P, P)], ssem[b])
                for b in range(B)
            ]
            pos_start(0, 0)
            pos_start(1, 1)
            for cp in idx_cps:
                cp.wait()
            for k in range(L):
                gather_start(k % NB, k % B, k // B)

            @pl.loop(0, NCH // 2)
            def _(it):
                c0 = it * 2
                for j in range(NB):
                    p, b = j // B, j % B
                    if b == 0:
                        drain(pos_v[p], psem[p])
                    j2 = (j + L) % NB
                    if j < NB - L:
                        @pl.when(it >= 1)
                        def _(_j2=j2):
                            drain(rows[_j2], ssem[_j2])
                    else:
                        drain(rows[j2], ssem[j2])
                    gather_start(j2, (j + L) % B,
                                 lax.min(c0 + (j + L) // B, NCH - 1))
                    drain(rows[j], gsem[j])

                    @pl.loop(0, W)
                    def _(r, _j=j, _p=p):
                        for col in range(0, H, LANES):
                            slc = (pl.ds(r, 1), pl.ds(col, LANES))
                            plsc.addupdate(rows[_j].at[*slc],
                                           pos_v[_p].at[*slc][...])

                    pltpu.async_copy(
                        rows[j],
                        out_hbm.at[pl.ds(b * S + pbase + (c0 + p) * W, W)],
                        ssem[j])
                    if b == B - 1:
                        pos_start(p, lax.min(c0 + p + 2, NCH - 1))

            for k in range(NIT - NB + L, NIT):
                drain(rows[k % NB], ssem[k % NB])
            for j in range(L):
                drain(rows[j], gsem[j])
            for p in range(2):
                drain(pos_v[p], psem[p])

        return emb_kernel(ids, tok, pos)

    out = run(ids, tok_embed, pos_embed)
    return out.reshape(B, S, H)


# speedup vs baseline: 1.2096x; 1.2096x over previous
import jax
import jax.numpy as jnp
from jax import lax
from jax.experimental import pallas as pl
from jax.experimental.pallas import tpu as pltpu
from jax.experimental.pallas import tpu_sc as plsc

LANES = 16
W = 16
NB = 8
L = 6


def kernel(input_ids, tok_embed, pos_embed):
    B, S = input_ids.shape
    H = tok_embed.shape[1]
    n = B * S
    NW = 32
    P = S // NW
    NCH = P // W
    NIT = NCH * B
    ids = input_ids.reshape(n).astype(jnp.int32)

    mesh = plsc.VectorSubcoreMesh(core_axis_name="c", subcore_axis_name="s")

    @jax.jit
    def run(ids, tok, pos):
        @pl.kernel(
            out_type=jax.ShapeDtypeStruct((n, H), jnp.float32),
            mesh=mesh,
            scratch_types=[pltpu.VMEM((B * P,), jnp.int32)]
            + [pltpu.VMEM((W, H), jnp.float32)] * 2
            + [pltpu.VMEM((W, H), jnp.float32)] * NB
            + [pltpu.SemaphoreType.DMA] * (2 + 2 * NB),
        )
        def emb_kernel(ids_hbm, tok_hbm, pos_hbm, out_hbm, idx_v, *rest):
            pos_v = rest[0:2]
            rows = rest[2:2 + NB]
            psem = rest[2 + NB:4 + NB]
            gsem = rest[4 + NB:4 + 2 * NB]
            ssem = rest[4 + 2 * NB:]
            wid = lax.axis_index("s") * 2 + lax.axis_index("c")
            pbase = wid * P

            def gather_start(j, b, ch):
                pltpu.async_copy(
                    tok_hbm.at[idx_v.at[pl.ds(b * P + ch * W, W)]],
                    rows[j], gsem[j])

            def pos_start(p, ch):
                pltpu.async_copy(pos_hbm.at[pl.ds(pbase + ch * W, W)],
                                 pos_v[p], psem[p])

            def drain(dst, sem):
                pltpu.make_async_copy(pos_hbm.at[pl.ds(0, W)], dst, sem).wait()

            idx_cps = [
                pltpu.async_copy(ids_hbm.at[pl.ds(b * S + pbase, P)],
                                 idx_v.at[pl.ds(b * P, P)], ssem[b])
                for b in range(B)
            ]
            pos_start(0, 0)
            pos_start(1, 1)
            for cp in idx_cps:
                cp.wait()
            for k in range(L):
                gather_start(k % NB, k % B, k // B)

            @pl.loop(0, NCH // 2)
            def _(it):
                c0 = it * 2
                for j in range(NB):
                    p, b = j // B, j % B
                    if b == 0:
                        drain(pos_v[p], psem[p])
                    j2 = (j + L) % NB
                    if j < NB - L:
                        @pl.when(it >= 1)
                        def _(_j2=j2):
                            drain(rows[_j2], ssem[_j2])
                    else:
                        drain(rows[j2], ssem[j2])
                    gather_start(j2, (j + L) % B,
                                 lax.min(c0 + (j + L) // B, NCH - 1))
                    drain(rows[j], gsem[j])

                    @pl.loop(0, W)
                    def _(r, _j=j, _p=p):
                        for col in range(0, H, LANES):
                            slc = (pl.ds(r, 1), pl.ds(col, LANES))
                            plsc.addupdate(rows[_j].at[*slc],
                                           pos_v[_p].at[*slc][...])

                    pltpu.async_copy(
                        rows[j],
                        out_hbm.at[pl.ds(b * S + pbase + (c0 + p) * W, W)],
                        ssem[j])
                    if b == B - 1:
                        pos_start(p, lax.min(c0 + p + 2, NCH - 1))

            for k in range(NIT - NB + L, NIT):
                drain(rows[k % NB], ssem[k % NB])
            for j in range(L):
                drain(rows[j], gsem[j])
            for p in range(2):
                drain(pos_v[p], psem[p])

        return emb_kernel(ids, tok, pos)

    out = run(ids, tok_embed, pos_embed)
    return out.reshape(B, S, H)
